# baseline (speedup 1.0000x reference)
import jax
import jax.numpy as jnp
from jax import lax
from jax.experimental import pallas as pl
from jax.experimental.pallas import tpu as pltpu, tpu_sc as plsc

BATCH = 4096
HIST_LEN = 200
DIM = 32
NUM_CORES = 2
NUM_SUBCORES = 16
NUM_WORKERS = NUM_CORES * NUM_SUBCORES
ROWS_PER_WORKER = BATCH // NUM_WORKERS
HALF = HIST_LEN // 2
NBUF = 4


NUM_EMB = 1000000
QSHIFT = 18
QUARTER = 1 << QSHIFT
QMASK = QUARTER - 1
TCOL = 16384
TGRID = QUARTER // TCOL
LASTB = (NUM_EMB - 1) // TCOL


def _untile_kernel(t0, t1, t2, t3, out_ref):
    stacked = jnp.concatenate(
        [t0[...], t1[...], t2[...], t3[...]], axis=0
    )
    out_ref[...] = jnp.transpose(stacked, (1, 0))


def _quarter_spec(a):
    return pl.BlockSpec(
        (DIM, TCOL), lambda i, a=a: (0, jnp.minimum(i + a * TGRID, LASTB))
    )


def _untile(tableT):
    return pl.pallas_call(
        _untile_kernel,
        out_shape=jax.ShapeDtypeStruct((QUARTER, 4 * DIM), jnp.float32),
        grid=(TGRID,),
        in_specs=[_quarter_spec(a) for a in range(4)],
        out_specs=pl.BlockSpec((TCOL, 4 * DIM), lambda i: (i, 0)),
        compiler_params=pltpu.CompilerParams(
            dimension_semantics=("parallel",)
        ),
    )(tableT, tableT, tableT, tableT)


def _emb_pool_kernel(x_hbm, table1d_hbm, out_hbm, idx_v, rows_v, out_v, *sems):
    table_hbm = table1d_hbm
    wid = lax.axis_index("s") * NUM_CORES + lax.axis_index("c")
    base = wid * ROWS_PER_WORKER

    pltpu.sync_copy(x_hbm.at[pl.ds(base * 2, ROWS_PER_WORKER * 2), :], idx_v)

    def issue(r, b):
        cp0 = pltpu.async_copy(table_hbm.at[idx_v.at[2 * r]], rows_v.at[b, 0], sems[b])
        cp1 = pltpu.async_copy(table_hbm.at[idx_v.at[2 * r + 1]], rows_v.at[b, 1], sems[b])
        return cp0, cp1

    def drain(r, b):
        pltpu.make_async_copy(table_hbm.at[idx_v.at[2 * r]], rows_v.at[b, 0], sems[b]).wait()
        pltpu.make_async_copy(table_hbm.at[idx_v.at[2 * r + 1]], rows_v.at[b, 1], sems[b]).wait()

    def accumulate(r, b):
        zero = jnp.zeros((16,), jnp.float32)
        half2 = HALF // 2

        def acc_body(j, carry):
            a0, a1, a2, a3, b0, b1, b2, b3 = carry
            a0 = a0 + rows_v[b, 0, j, pl.ds(0, 16)]
            a1 = a1 + rows_v[b, 0, j, pl.ds(16, 16)]
            a2 = a2 + rows_v[b, 0, half2 + j, pl.ds(0, 16)]
            a3 = a3 + rows_v[b, 0, half2 + j, pl.ds(16, 16)]
            b0 = b0 + rows_v[b, 1, j, pl.ds(0, 16)]
            b1 = b1 + rows_v[b, 1, j, pl.ds(16, 16)]
            b2 = b2 + rows_v[b, 1, half2 + j, pl.ds(0, 16)]
            b3 = b3 + rows_v[b, 1, half2 + j, pl.ds(16, 16)]
            return a0, a1, a2, a3, b0, b1, b2, b3

        a0, a1, a2, a3, b0, b1, b2, b3 = lax.fori_loop(
            0, half2, acc_body, (zero,) * 8, unroll=10
        )
        scale = jnp.float32(1.0 / HIST_LEN)
        out_v[r, pl.ds(0, 16)] = ((a0 + b0) + (a2 + b2)) * scale
        out_v[r, pl.ds(16, 16)] = ((a1 + b1) + (a3 + b3)) * scale

    for b in range(NBUF):
        issue(b, b)

    @pl.loop(0, ROWS_PER_WORKER - NBUF, step=NBUF)
    def _main(g):
        for b in range(NBUF):
            r = g + b
            drain(r, b)
            issue(r + NBUF, b)
            accumulate(r, b)

    for b in range(NBUF):
        r = ROWS_PER_WORKER - NBUF + b
        drain(r, b)
        accumulate(r, b)

    pltpu.sync_copy(out_v, out_hbm.at[pl.ds(base, ROWS_PER_WORKER), :])


@jax.jit
def kernel(x, table):
    mesh = plsc.VectorSubcoreMesh(core_axis_name="c", subcore_axis_name="s")
    x = x.astype(jnp.int32)
    xq = ((x & QMASK) << 2) | (x >> QSHIFT)
    x2 = xq.reshape(BATCH * 2, HALF)
    run = pl.kernel(
        _emb_pool_kernel,
        out_type=jax.ShapeDtypeStruct((BATCH, DIM), jnp.float32),
        mesh=mesh,
        scratch_types=[
            pltpu.VMEM((ROWS_PER_WORKER * 2, HALF), jnp.int32),
            pltpu.VMEM((NBUF, 2, HALF, DIM), jnp.float32),
            pltpu.VMEM((ROWS_PER_WORKER, DIM), jnp.float32),
        ]
        + [pltpu.SemaphoreType.DMA] * NBUF,
        compiler_params=pltpu.CompilerParams(use_tc_tiling_on_sc=False),
    )
    t2 = _untile(table.T).reshape(QUARTER * 4, DIM)
    return run(x2, t2)


# speedup vs baseline: 1.0226x; 1.0226x over previous
import jax
import jax.numpy as jnp
from jax import lax
from jax.experimental import pallas as pl
from jax.experimental.pallas import tpu as pltpu, tpu_sc as plsc

BATCH = 4096
HIST_LEN = 200
DIM = 32
NUM_CORES = 2
NUM_SUBCORES = 16
NUM_WORKERS = NUM_CORES * NUM_SUBCORES
ROWS_PER_WORKER = BATCH // NUM_WORKERS
HALF = HIST_LEN // 2
NBUF = 8


NUM_EMB = 1000000
QSHIFT = 18
QUARTER = 1 << QSHIFT
QMASK = QUARTER - 1
TCOL = 16384
TGRID = QUARTER // TCOL
LASTB = (NUM_EMB - 1) // TCOL


def _untile_kernel(t0, t1, t2, t3, out_ref):
    stacked = jnp.concatenate(
        [t0[...], t1[...], t2[...], t3[...]], axis=0
    )
    out_ref[...] = jnp.transpose(stacked, (1, 0))


def _quarter_spec(a):
    return pl.BlockSpec(
        (DIM, TCOL), lambda i, a=a: (0, jnp.minimum(i + a * TGRID, LASTB))
    )


def _untile(tableT):
    return pl.pallas_call(
        _untile_kernel,
        out_shape=jax.ShapeDtypeStruct((QUARTER, 4 * DIM), jnp.float32),
        grid=(TGRID,),
        in_specs=[_quarter_spec(a) for a in range(4)],
        out_specs=pl.BlockSpec((TCOL, 4 * DIM), lambda i: (i, 0)),
        compiler_params=pltpu.CompilerParams(
            dimension_semantics=("parallel",)
        ),
    )(tableT, tableT, tableT, tableT)


def _emb_pool_kernel(x_hbm, table1d_hbm, out_hbm, idx_v, rows_v, out_v, *sems):
    table_hbm = table1d_hbm
    wid = lax.axis_index("s") * NUM_CORES + lax.axis_index("c")
    base = wid * ROWS_PER_WORKER

    pltpu.sync_copy(x_hbm.at[pl.ds(base * 2, ROWS_PER_WORKER * 2), :], idx_v)

    def issue(r, b):
        cp0 = pltpu.async_copy(table_hbm.at[idx_v.at[2 * r]], rows_v.at[b, 0], sems[b])
        cp1 = pltpu.async_copy(table_hbm.at[idx_v.at[2 * r + 1]], rows_v.at[b, 1], sems[b])
        return cp0, cp1

    def drain(r, b):
        pltpu.make_async_copy(table_hbm.at[idx_v.at[2 * r]], rows_v.at[b, 0], sems[b]).wait()
        pltpu.make_async_copy(table_hbm.at[idx_v.at[2 * r + 1]], rows_v.at[b, 1], sems[b]).wait()

    def accumulate(r, b):
        zero = jnp.zeros((16,), jnp.float32)

        def acc_body(j, carry):
            a0, a1, b0, b1 = carry
            a0 = a0 + rows_v[b, 0, j, pl.ds(0, 16)]
            a1 = a1 + rows_v[b, 0, j, pl.ds(16, 16)]
            b0 = b0 + rows_v[b, 1, j, pl.ds(0, 16)]
            b1 = b1 + rows_v[b, 1, j, pl.ds(16, 16)]
            return a0, a1, b0, b1

        a0, a1, b0, b1 = lax.fori_loop(
            0, HALF, acc_body, (zero, zero, zero, zero), unroll=10
        )
        scale = jnp.float32(1.0 / HIST_LEN)
        out_v[r, pl.ds(0, 16)] = (a0 + b0) * scale
        out_v[r, pl.ds(16, 16)] = (a1 + b1) * scale

    for b in range(NBUF):
        issue(b, b)

    @pl.loop(0, ROWS_PER_WORKER - NBUF, step=NBUF)
    def _main(g):
        for b in range(NBUF):
            r = g + b
            drain(r, b)
            issue(r + NBUF, b)
            accumulate(r, b)

    for b in range(NBUF):
        r = ROWS_PER_WORKER - NBUF + b
        drain(r, b)
        accumulate(r, b)

    pltpu.sync_copy(out_v, out_hbm.at[pl.ds(base, ROWS_PER_WORKER), :])


@jax.jit
def kernel(x, table):
    mesh = plsc.VectorSubcoreMesh(core_axis_name="c", subcore_axis_name="s")
    x = x.astype(jnp.int32)
    xq = ((x & QMASK) << 2) | (x >> QSHIFT)
    x2 = xq.reshape(BATCH * 2, HALF)
    run = pl.kernel(
        _emb_pool_kernel,
        out_type=jax.ShapeDtypeStruct((BATCH, DIM), jnp.float32),
        mesh=mesh,
        scratch_types=[
            pltpu.VMEM((ROWS_PER_WORKER * 2, HALF), jnp.int32),
            pltpu.VMEM((NBUF, 2, HALF, DIM), jnp.float32),
            pltpu.VMEM((ROWS_PER_WORKER, DIM), jnp.float32),
        ]
        + [pltpu.SemaphoreType.DMA] * NBUF,
        compiler_params=pltpu.CompilerParams(use_tc_tiling_on_sc=False),
    )
    t2 = _untile(table.T).reshape(QUARTER * 4, DIM)
    return run(x2, t2)


# speedup vs baseline: 1.0402x; 1.0172x over previous
import jax
import jax.numpy as jnp
from jax import lax
from jax.experimental import pallas as pl
from jax.experimental.pallas import tpu as pltpu, tpu_sc as plsc

BATCH = 4096
HIST_LEN = 200
DIM = 32
NUM_CORES = 2
NUM_SUBCORES = 16
NUM_WORKERS = NUM_CORES * NUM_SUBCORES
ROWS_PER_WORKER = BATCH // NUM_WORKERS
HALF = HIST_LEN // 2
SPLIT0 = 104
SPLIT1 = HIST_LEN - SPLIT0
NBUF = 8


NUM_EMB = 1000000
QSHIFT = 18
QUARTER = 1 << QSHIFT
QMASK = QUARTER - 1
TCOL = 16384
TGRID = QUARTER // TCOL
LASTB = (NUM_EMB - 1) // TCOL


def _untile_kernel(t0, t1, t2, t3, out_ref):
    stacked = jnp.concatenate(
        [t0[...], t1[...], t2[...], t3[...]], axis=0
    )
    out_ref[...] = jnp.transpose(stacked, (1, 0))


def _quarter_spec(a):
    return pl.BlockSpec(
        (DIM, TCOL), lambda i, a=a: (0, jnp.minimum(i + a * TGRID, LASTB))
    )


def _untile(tableT):
    return pl.pallas_call(
        _untile_kernel,
        out_shape=jax.ShapeDtypeStruct((QUARTER, 4 * DIM), jnp.float32),
        grid=(TGRID,),
        in_specs=[_quarter_spec(a) for a in range(4)],
        out_specs=pl.BlockSpec((TCOL, 4 * DIM), lambda i: (i, 0)),
        compiler_params=pltpu.CompilerParams(
            dimension_semantics=("parallel",)
        ),
    )(tableT, tableT, tableT, tableT)


def _emb_pool_kernel(x_hbm, table1d_hbm, out_hbm, idx_v, rows_v, out_v, *sems):
    table_hbm = table1d_hbm
    wid = lax.axis_index("s") * NUM_CORES + lax.axis_index("c")
    base = wid * ROWS_PER_WORKER

    pltpu.sync_copy(
        x_hbm.at[pl.ds(base * HIST_LEN, ROWS_PER_WORKER * HIST_LEN)], idx_v
    )

    def issue(r, b):
        cp0 = pltpu.async_copy(
            table_hbm.at[idx_v.at[pl.ds(r * HIST_LEN, SPLIT0)]], rows_v.at[b, 0], sems[b]
        )
        cp1 = pltpu.async_copy(
            table_hbm.at[idx_v.at[pl.ds(r * HIST_LEN + SPLIT0, SPLIT1)]],
            rows_v.at[b, 1, pl.ds(0, SPLIT1)],
            sems[b],
        )
        return cp0, cp1

    def drain(r, b):
        pltpu.make_async_copy(
            table_hbm.at[idx_v.at[pl.ds(r * HIST_LEN, SPLIT0)]], rows_v.at[b, 0], sems[b]
        ).wait()
        pltpu.make_async_copy(
            table_hbm.at[idx_v.at[pl.ds(r * HIST_LEN + SPLIT0, SPLIT1)]],
            rows_v.at[b, 1, pl.ds(0, SPLIT1)],
            sems[b],
        ).wait()

    def accumulate(r, b):
        zero = jnp.zeros((16,), jnp.float32)

        def acc_body(j, carry):
            a0, a1, b0, b1 = carry
            a0 = a0 + rows_v[b, 0, j, pl.ds(0, 16)]
            a1 = a1 + rows_v[b, 0, j, pl.ds(16, 16)]
            b0 = b0 + rows_v[b, 1, j, pl.ds(0, 16)]
            b1 = b1 + rows_v[b, 1, j, pl.ds(16, 16)]
            return a0, a1, b0, b1

        a0, a1, b0, b1 = lax.fori_loop(
            0, SPLIT1, acc_body, (zero, zero, zero, zero), unroll=8
        )

        def tail_body(j, carry):
            t0, t1 = carry
            t0 = t0 + rows_v[b, 0, j, pl.ds(0, 16)]
            t1 = t1 + rows_v[b, 0, j, pl.ds(16, 16)]
            return t0, t1

        a0, a1 = lax.fori_loop(SPLIT1, SPLIT0, tail_body, (a0, a1), unroll=8)
        scale = jnp.float32(1.0 / HIST_LEN)
        out_v[r, pl.ds(0, 16)] = (a0 + b0) * scale
        out_v[r, pl.ds(16, 16)] = (a1 + b1) * scale

    for b in range(NBUF):
        issue(b, b)

    @pl.loop(0, ROWS_PER_WORKER - NBUF, step=NBUF)
    def _main(g):
        for b in range(NBUF):
            r = g + b
            drain(r, b)
            issue(r + NBUF, b)
            accumulate(r, b)

    for b in range(NBUF):
        r = ROWS_PER_WORKER - NBUF + b
        drain(r, b)
        accumulate(r, b)

    pltpu.sync_copy(out_v, out_hbm.at[pl.ds(base, ROWS_PER_WORKER), :])


@jax.jit
def kernel(x, table):
    mesh = plsc.VectorSubcoreMesh(core_axis_name="c", subcore_axis_name="s")
    x = x.astype(jnp.int32)
    xq = ((x & QMASK) << 2) | (x >> QSHIFT)
    x2 = xq.reshape(BATCH * HIST_LEN)
    run = pl.kernel(
        _emb_pool_kernel,
        out_type=jax.ShapeDtypeStruct((BATCH, DIM), jnp.float32),
        mesh=mesh,
        scratch_types=[
            pltpu.VMEM((ROWS_PER_WORKER * HIST_LEN,), jnp.int32),
            pltpu.VMEM((NBUF, 2, SPLIT0, DIM), jnp.float32),
            pltpu.VMEM((ROWS_PER_WORKER, DIM), jnp.float32),
        ]
        + [pltpu.SemaphoreType.DMA] * NBUF,
        compiler_params=pltpu.CompilerParams(use_tc_tiling_on_sc=False),
    )
    t2 = _untile(table.T).reshape(QUARTER * 4, DIM)
    return run(x2, t2)
